# Initial kernel scaffold; baseline (speedup 1.0000x reference)
import jax
import jax.numpy as jnp
from jax.experimental import pallas as pl


def kernel(pointcloud, params):
    raise NotImplementedError("write your pallas kernel here")


# speedup vs baseline: 13.0164x; 13.0164x over previous
import functools

import jax
import jax.numpy as jnp
from jax import lax
from jax.experimental import pallas as pl

F32 = jnp.float32

_FPS_LEVELS = (1024, 256, 64, 16)


def _mlp_layer_body(y_ref, w_ref, scale_ref, bias_ref, out_ref, stats_ref,
                    *, first):
    y = y_ref[...]
    if first:
        h = y
    else:
        h = jnp.maximum(y * scale_ref[...] + bias_ref[...], 0.0)
    y2 = lax.dot_general(h, w_ref[...], (((1,), (1,)), ((), ())),
                         preferred_element_type=F32,
                         precision=lax.Precision.HIGHEST)
    out_ref[...] = y2
    s1 = jnp.sum(y2, axis=0)[None, :]
    s2 = jnp.sum(y2 * y2, axis=0)[None, :]
    s = jnp.concatenate([s1, s2], axis=0)

    @pl.when(pl.program_id(0) == 0)
    def _():
        stats_ref[...] = s

    @pl.when(pl.program_id(0) > 0)
    def _():
        stats_ref[...] = stats_ref[...] + s


def _mlp_layer(y, w, scale, bias, tm, first=False):
    m, cin = y.shape
    cout = w.shape[0]
    body = functools.partial(_mlp_layer_body, first=first)
    return pl.pallas_call(
        body,
        grid=(m // tm,),
        in_specs=[
            pl.BlockSpec((tm, cin), lambda i: (i, 0)),
            pl.BlockSpec((cout, cin), lambda i: (0, 0)),
            pl.BlockSpec((1, cin), lambda i: (0, 0)),
            pl.BlockSpec((1, cin), lambda i: (0, 0)),
        ],
        out_specs=[
            pl.BlockSpec((tm, cout), lambda i: (i, 0)),
            pl.BlockSpec((2, cout), lambda i: (0, 0)),
        ],
        out_shape=[
            jax.ShapeDtypeStruct((m, cout), F32),
            jax.ShapeDtypeStruct((2, cout), F32),
        ],
    )(y, w, scale, bias)


def _mean_body(y_ref, scale_ref, bias_ref, out_ref, *, blocks_per_batch):
    i = pl.program_id(0)
    h = jnp.maximum(y_ref[...] * scale_ref[...] + bias_ref[...], 0.0)
    s = jnp.sum(h, axis=0)[None, :]

    @pl.when(i == 0)
    def _():
        out_ref[...] = jnp.zeros_like(out_ref)

    b = i // blocks_per_batch
    out_ref[pl.ds(b, 1), :] = out_ref[pl.ds(b, 1), :] + s


def _mlp_mean(y, scale, bias, batch, rows_per_batch, tm):
    m, c = y.shape
    body = functools.partial(_mean_body, blocks_per_batch=rows_per_batch // tm)
    return pl.pallas_call(
        body,
        grid=(m // tm,),
        in_specs=[
            pl.BlockSpec((tm, c), lambda i: (i, 0)),
            pl.BlockSpec((1, c), lambda i: (0, 0)),
            pl.BlockSpec((1, c), lambda i: (0, 0)),
        ],
        out_specs=pl.BlockSpec((batch, c), lambda i: (0, 0)),
        out_shape=jax.ShapeDtypeStruct((batch, c), F32),
    )(y, scale, bias)


def _affine_coeffs(stats, gamma, beta, m):
    mean = stats[0] / m
    var = stats[1] / m - mean * mean
    rstd = 1.0 / jnp.sqrt(var + 1e-5)
    scale = gamma * rstd
    bias = beta - mean * scale
    return scale.reshape(1, -1), bias.reshape(1, -1)


def _fps_level(x, y, z, npoint):
    b, n = x.shape
    iota = lax.broadcasted_iota(jnp.int32, (b, n), 1)
    sel_iota = lax.broadcasted_iota(jnp.int32, (b, npoint), 1)
    bx, by, bz = x[:, 0:1], y[:, 0:1], z[:, 0:1]
    sx = jnp.where(sel_iota == 0, bx, 0.0)
    sy = jnp.where(sel_iota == 0, by, 0.0)
    sz = jnp.where(sel_iota == 0, bz, 0.0)
    dist = jnp.full((b, n), 1e10, dtype=F32)

    def body(i, carry):
        dist, bx, by, bz, sx, sy, sz = carry
        dx = x - bx
        dy = y - by
        dz = z - bz
        d = (dx * dx + dy * dy) + dz * dz
        dist = jnp.minimum(dist, d)
        mx = jnp.max(dist, axis=1, keepdims=True)
        idx = jnp.min(jnp.where(dist == mx, iota, n), axis=1, keepdims=True)
        onehot = iota == idx
        nbx = jnp.sum(jnp.where(onehot, x, 0.0), axis=1, keepdims=True)
        nby = jnp.sum(jnp.where(onehot, y, 0.0), axis=1, keepdims=True)
        nbz = jnp.sum(jnp.where(onehot, z, 0.0), axis=1, keepdims=True)
        sx = jnp.where(sel_iota == i, nbx, sx)
        sy = jnp.where(sel_iota == i, nby, sy)
        sz = jnp.where(sel_iota == i, nbz, sz)
        return dist, nbx, nby, nbz, sx, sy, sz

    carry = lax.fori_loop(1, npoint, body,
                          (dist, bx, by, bz, sx, sy, sz))
    return carry[4], carry[5], carry[6]


def _rbf_mean(x, y, z, c_ref, ls_ref):
    cx = c_ref[0:1, :][:, None, :]
    cy = c_ref[1:2, :][:, None, :]
    cz = c_ref[2:3, :][:, None, :]
    dx = x[:, :, None] - cx
    dy = y[:, :, None] - cy
    dz = z[:, :, None] - cz
    d2 = (dx * dx + dy * dy) + dz * dz
    sig = jnp.exp(ls_ref[...])[:, None, :]
    d = jnp.sqrt(jnp.maximum(d2, 0.0) + 1e-12) * sig
    return jnp.mean(jnp.exp(-(d * d)), axis=1)


def _geo_body(xyzT_ref, c2_ref, ls2_ref, c3_ref, ls3_ref, c4_ref, ls4_ref,
              r2_ref, r3_ref, r4_ref):
    x0 = xyzT_ref[0]
    y0 = xyzT_ref[1]
    z0 = xyzT_ref[2]
    x1, y1, z1 = _fps_level(x0, y0, z0, _FPS_LEVELS[0])
    x2, y2, z2 = _fps_level(x1, y1, z1, _FPS_LEVELS[1])
    x3, y3, z3 = _fps_level(x2, y2, z2, _FPS_LEVELS[2])
    x4, y4, z4 = _fps_level(x3, y3, z3, _FPS_LEVELS[3])
    r2_ref[...] = _rbf_mean(x2, y2, z2, c2_ref, ls2_ref)
    r3_ref[...] = _rbf_mean(x3, y3, z3, c3_ref, ls3_ref)
    r4_ref[...] = _rbf_mean(x4, y4, z4, c4_ref, ls4_ref)


def _geo(pointcloud, rbf_params):
    b = pointcloud.shape[0]
    xyzT = jnp.transpose(pointcloud, (2, 0, 1))
    args = [xyzT]
    for lvl in (2, 3, 4):
        args.append(rbf_params[lvl]['centres'].T)
        args.append(rbf_params[lvl]['log_sigmas'].reshape(1, -1))
    ks = [rbf_params[lvl]['centres'].shape[0] for lvl in (2, 3, 4)]
    return pl.pallas_call(
        _geo_body,
        out_shape=[jax.ShapeDtypeStruct((b, k), F32) for k in ks],
    )(*args)


def kernel(pointcloud, params):
    b, n, _ = pointcloud.shape
    m = b * n
    tm = 2048

    x0 = pointcloud.reshape(m, 3)
    gl = params['global']
    dummy = jnp.zeros((1, x0.shape[1]), F32)
    y, stats = _mlp_layer(x0, gl[0]['W'], dummy, dummy, tm, first=True)
    for i in range(1, 6):
        scale, bias = _affine_coeffs(stats, gl[i - 1]['gamma'],
                                     gl[i - 1]['beta'], m)
        y, stats = _mlp_layer(y, gl[i]['W'], scale, bias, tm)
    scale, bias = _affine_coeffs(stats, gl[5]['gamma'], gl[5]['beta'], m)
    gsum = _mlp_mean(y, scale, bias, b, n, tm)
    g_features = gsum * (1.0 / n)

    r2, r3, r4 = _geo(pointcloud, params['rbf'])
    return jnp.concatenate([g_features, r2, r3, r4], axis=1)
